# baseline (speedup 1.0000x reference)
import functools

import jax
import jax.numpy as jnp
from jax import lax
from jax.experimental import pallas as pl
from jax.experimental.pallas import tpu as pltpu
from jax.experimental.pallas import tpu_sc as plsc

_T = 128
_N = 32
_D = 3 * 224 * 224


def _src_frame(w):
    bump = lambda k: jnp.where(w >= k, jnp.int32(1), jnp.int32(0))
    return jnp.int32(4) * w + bump(6) + bump(16) + bump(26)


def _sc_subsample(x2):
    mesh = plsc.VectorSubcoreMesh(core_axis_name="c", subcore_axis_name="s")

    @functools.partial(
        pl.kernel,
        mesh=mesh,
        out_type=jax.ShapeDtypeStruct((_N, _D), jnp.float32),
    )
    def body(x_hbm, out_hbm):
        w = lax.axis_index("s") * 2 + lax.axis_index("c")
        f = _src_frame(w)
        pltpu.sync_copy(x_hbm.at[f], out_hbm.at[w])

    return body(x2)


def kernel(x):
    x2 = x.reshape(_T, _D)
    out = _sc_subsample(x2)
    return out.reshape(_N, 3, 224, 224)


# speedup vs baseline: 5.4268x; 5.4268x over previous
import functools

import jax
import jax.numpy as jnp
from jax import lax
from jax.experimental import pallas as pl
from jax.experimental.pallas import tpu as pltpu
from jax.experimental.pallas import tpu_sc as plsc

_T = 128
_N = 32
_D = 3 * 224 * 224


def _src_frame(w):
    bump = lambda k: jnp.where(w >= k, jnp.int32(1), jnp.int32(0))
    return jnp.int32(4) * w + bump(6) + bump(16) + bump(26)


_NCH = 4
_CH = _D // _NCH


def _sc_subsample(x2):
    mesh = plsc.VectorSubcoreMesh(core_axis_name="c", subcore_axis_name="s")

    @functools.partial(
        pl.kernel,
        mesh=mesh,
        out_type=jax.ShapeDtypeStruct((_N, _D), jnp.float32),
        scratch_types=[
            pltpu.VMEM((_CH,), jnp.float32),
            pltpu.VMEM((_CH,), jnp.float32),
            pltpu.VMEM((_CH,), jnp.float32),
            pltpu.SemaphoreType.DMA,
            pltpu.SemaphoreType.DMA,
            pltpu.SemaphoreType.DMA,
            pltpu.SemaphoreType.DMA,
            pltpu.SemaphoreType.DMA,
            pltpu.SemaphoreType.DMA,
        ],
    )
    def body(x_hbm, out_hbm, b0, b1, b2, g0, g1, g2, s0, s1, s2):
        w = lax.axis_index("s") * 2 + lax.axis_index("c")
        f = _src_frame(w)
        bufs = (b0, b1, b2)
        gsems = (g0, g1, g2)
        ssems = (s0, s1, s2)

        gathers = [None] * _NCH
        scatters = [None] * _NCH
        for c in range(min(3, _NCH)):
            gathers[c] = pltpu.async_copy(
                x_hbm.at[f, pl.ds(c * _CH, _CH)], bufs[c], gsems[c]
            )
        for c in range(_NCH):
            b = c % 3
            gathers[c].wait()
            scatters[c] = pltpu.async_copy(
                bufs[b], out_hbm.at[w, pl.ds(c * _CH, _CH)], ssems[b]
            )
            nxt = c + 3
            if nxt < _NCH:
                scatters[c].wait()
                gathers[nxt] = pltpu.async_copy(
                    x_hbm.at[f, pl.ds(nxt * _CH, _CH)], bufs[b], gsems[b]
                )
        for c in range(max(0, _NCH - 3), _NCH):
            if scatters[c] is not None and c + 3 >= _NCH:
                scatters[c].wait()

    return body(x2)


def kernel(x):
    x2 = x.reshape(_T, _D)
    out = _sc_subsample(x2)
    return out.reshape(_N, 3, 224, 224)


# speedup vs baseline: 6.2716x; 1.1557x over previous
import functools

import jax
import jax.numpy as jnp
from jax import lax
from jax.experimental import pallas as pl
from jax.experimental.pallas import tpu as pltpu
from jax.experimental.pallas import tpu_sc as plsc

_T = 128
_N = 32
_C = 3
_H = 224
_W = 224


def _src_frame(w):
    bump = lambda k: jnp.where(w >= k, jnp.int32(1), jnp.int32(0))
    return jnp.int32(4) * w + bump(6) + bump(16) + bump(26)


def _sc_subsample(x):
    mesh = plsc.VectorSubcoreMesh(core_axis_name="c", subcore_axis_name="s")

    @functools.partial(
        pl.kernel,
        mesh=mesh,
        out_type=jax.ShapeDtypeStruct((_N, _C, _H, _W), jnp.float32),
        scratch_types=[
            pltpu.VMEM((_H, _W), jnp.float32),
            pltpu.VMEM((_H, _W), jnp.float32),
            pltpu.SemaphoreType.DMA,
            pltpu.SemaphoreType.DMA,
            pltpu.SemaphoreType.DMA,
            pltpu.SemaphoreType.DMA,
        ],
    )
    def body(x_hbm, out_hbm, b0, b1, g0, g1, s0, s1):
        w = lax.axis_index("s") * 2 + lax.axis_index("c")
        f = _src_frame(w)
        bufs = (b0, b1)
        gsems = (g0, g1)
        ssems = (s0, s1)

        gathers = [None] * _C
        scatters = [None] * _C
        for c in range(min(2, _C)):
            gathers[c] = pltpu.async_copy(x_hbm.at[f, c], bufs[c], gsems[c])
        for c in range(_C):
            b = c % 2
            gathers[c].wait()
            scatters[c] = pltpu.async_copy(bufs[b], out_hbm.at[w, c], ssems[b])
            nxt = c + 2
            if nxt < _C:
                scatters[c].wait()
                gathers[nxt] = pltpu.async_copy(
                    x_hbm.at[f, nxt], bufs[b], gsems[b]
                )
        for c in range(_C):
            if c + 2 >= _C:
                scatters[c].wait()

    return body(x)


def kernel(x):
    return _sc_subsample(x)
